# Initial kernel scaffold; baseline (speedup 1.0000x reference)
import jax
import jax.numpy as jnp
from jax.experimental import pallas as pl


def kernel(positions, velocities, masses):
    raise NotImplementedError("write your pallas kernel here")


# speedup vs baseline: 1.2873x; 1.2873x over previous
import functools

import jax
import jax.numpy as jnp
from jax import lax
from jax.experimental import pallas as pl
from jax.experimental.pallas import tpu as pltpu
from jax.experimental.pallas import tpu_sc as plsc

_R_BINS = 32
_N = 4_000_000
_NC, _NS, _L = 2, 16, 16
_NW = _NC * _NS
_CHUNK = 4000
_NCHUNKS = _N // _CHUNK
_CPW = (_NCHUNKS + _NW - 1) // _NW
_GROUPS = _CHUNK // _L
_NVAL = 7
_HIST = _NVAL * _R_BINS * _L
_ROWS = _NVAL * _R_BINS


def _sc_body(pos_hbm, vel_hbm, mass_hbm, out_hbm,
             pos_v, vel_v, m_v, hist_v, acc_v, sem):
    cid = lax.axis_index("c")
    sid = lax.axis_index("s")
    wid = sid * _NC + cid

    lane = lax.iota(jnp.int32, _L)
    iota3 = lane * 3
    zero16 = jnp.zeros((_L,), jnp.float32)

    def _zero(j, carry):
        hist_v[pl.ds(j * _L, _L)] = zero16
        return carry

    lax.fori_loop(0, _HIST // _L, _zero, 0)

    def chunk_body(t, carry):
        c = wid + t * _NW

        @pl.when(c < _NCHUNKS)
        def _():
            base = c * _CHUNK
            cp_p = pltpu.async_copy(
                pos_hbm.at[pl.ds(base * 3, _CHUNK * 3)], pos_v, sem)
            cp_v = pltpu.async_copy(
                vel_hbm.at[pl.ds(base * 3, _CHUNK * 3)], vel_v, sem)
            cp_m = pltpu.async_copy(
                mass_hbm.at[pl.ds(base, _CHUNK)], m_v, sem)
            cp_p.wait()
            cp_v.wait()
            cp_m.wait()

            def group_body(g, gcarry):
                ix = iota3 + g * (3 * _L)
                x = plsc.load_gather(pos_v, [ix])
                y = plsc.load_gather(pos_v, [ix + 1])
                vx = plsc.load_gather(vel_v, [ix])
                vy = plsc.load_gather(vel_v, [ix + 1])
                vz = plsc.load_gather(vel_v, [ix + 2])
                m = m_v[pl.ds(g * _L, _L)]

                s = x * x + y * y
                inv = plsc.bitcast(
                    jnp.int32(0x5F3759DF) - (plsc.bitcast(s, jnp.int32) >> 1),
                    jnp.float32)
                h = -0.5 * s
                inv = inv * (1.5 + h * inv * inv)
                inv = inv * (1.5 + h * inv * inv)
                inv = inv * (1.5 + h * inv * inv)

                u = (s * inv) * 8.0
                i0 = u.astype(jnp.int32)
                fi = i0.astype(jnp.float32)
                s64 = s * 64.0
                fh = fi + 1.0
                i1 = jnp.where(s64 >= fh * fh, i0 + 1, i0)
                i1 = jnp.where(s64 < fi * fi, i1 - 1, i1)
                w = jnp.where(i1 < _R_BINS, m, 0.0)
                ic = jnp.minimum(i1, _R_BINS - 1)

                nr = x * vx + y * vy
                nphi = y * vx - x * vy
                vr = nr * inv
                vphi = nphi * inv
                wvr = w * vr
                wvphi = w * vphi
                wvz = w * vz
                bidx = ic * _L + lane
                plsc.addupdate_scatter(hist_v, [bidx], w)
                plsc.addupdate_scatter(hist_v, [bidx + 512], wvr)
                plsc.addupdate_scatter(hist_v, [bidx + 1024], wvr * vr)
                plsc.addupdate_scatter(hist_v, [bidx + 1536], wvphi)
                plsc.addupdate_scatter(hist_v, [bidx + 2048], wvphi * vphi)
                plsc.addupdate_scatter(hist_v, [bidx + 2560], wvz)
                plsc.addupdate_scatter(hist_v, [bidx + 3072], wvz * vz)
                return gcarry

            lax.fori_loop(0, _GROUPS, group_body, 0)

        return carry

    lax.fori_loop(0, _CPW, chunk_body, 0)

    lidx = lane * _L
    for j in range(_ROWS // _L):
        b = j * (_L * _L)
        accv = zero16
        for l in range(_L):
            accv = accv + plsc.load_gather(hist_v, [lidx + (b + l)])
        acc_v[pl.ds(j * _L, _L)] = accv

    pltpu.sync_copy(acc_v, out_hbm.at[wid])


_sc_hist = functools.partial(
    pl.kernel,
    out_type=jax.ShapeDtypeStruct((_NW, _ROWS), jnp.float32),
    mesh=plsc.VectorSubcoreMesh(
        core_axis_name="c", subcore_axis_name="s",
        num_cores=_NC, num_subcores=_NS),
    compiler_params=pltpu.CompilerParams(needs_layout_passes=False),
    scratch_types=[
        pltpu.VMEM((_CHUNK * 3,), jnp.float32),
        pltpu.VMEM((_CHUNK * 3,), jnp.float32),
        pltpu.VMEM((_CHUNK,), jnp.float32),
        pltpu.VMEM((_HIST,), jnp.float32),
        pltpu.VMEM((_ROWS,), jnp.float32),
        pltpu.SemaphoreType.DMA,
    ],
)(_sc_body)


def _epi_body(p_ref, o_ref):
    s = jnp.sum(p_ref[:], axis=0)
    mass = s[0:1, :]
    vr = s[1:2] / mass
    vr2 = s[2:3] / mass
    vphi = s[3:4] / mass
    vphi2 = s[4:5] / mass
    vz = s[5:6] / mass
    vz2 = s[6:7] / mass
    o_ref[:] = jnp.concatenate([
        vphi, jnp.sqrt(vphi2 - vphi * vphi),
        vr, jnp.sqrt(vr2 - vr * vr),
        vz, jnp.sqrt(vz2 - vz * vz)], axis=0)


def kernel(positions, velocities, masses):
    partials = _sc_hist(positions.reshape(-1), velocities.reshape(-1), masses)
    p3 = partials.reshape(_NW, _NVAL, _R_BINS)
    return pl.pallas_call(
        _epi_body,
        out_shape=jax.ShapeDtypeStruct((6, _R_BINS), jnp.float32),
    )(p3)


# speedup vs baseline: 42.4544x; 32.9797x over previous
import functools

import jax
import jax.numpy as jnp
from jax import lax
from jax.experimental import pallas as pl
from jax.experimental.pallas import tpu as pltpu
from jax.experimental.pallas import tpu_sc as plsc

_R_BINS = 32
_N = 4_000_000
_NC, _NS, _L = 2, 16, 16
_NW = _NC * _NS
_CHUNK = 8000
_NCHUNKS = _N // _CHUNK
_CPW = (_NCHUNKS + _NW - 1) // _NW
_GROUPS = _CHUNK // _L
_NVAL = 7
_HIST = _NVAL * _R_BINS * _L
_ROWS = _NVAL * _R_BINS


def _sc_body(x_hbm, y_hbm, vx_hbm, vy_hbm, vz_hbm, m_hbm, out_hbm,
             x_v, y_v, vx_v, vy_v, vz_v, m_v, hist_v, acc_v, sem):
    cid = lax.axis_index("c")
    sid = lax.axis_index("s")
    wid = sid * _NC + cid

    lane = lax.iota(jnp.int32, _L)
    zero16 = jnp.zeros((_L,), jnp.float32)

    def _zero(j, carry):
        hist_v[pl.ds(j * _L, _L)] = zero16
        return carry

    lax.fori_loop(0, _HIST // _L, _zero, 0)

    def chunk_body(t, carry):
        c = wid + t * _NW

        @pl.when(c < _NCHUNKS)
        def _():
            base = c * _CHUNK
            cps = [
                pltpu.async_copy(h.at[pl.ds(base, _CHUNK)], v, sem)
                for h, v in ((x_hbm, x_v), (y_hbm, y_v), (vx_hbm, vx_v),
                             (vy_hbm, vy_v), (vz_hbm, vz_v), (m_hbm, m_v))
            ]
            for cp in cps:
                cp.wait()

            def group_body(g, gcarry):
                sl = pl.ds(g * _L, _L)
                x = x_v[sl]
                y = y_v[sl]
                vx = vx_v[sl]
                vy = vy_v[sl]
                vz = vz_v[sl]
                m = m_v[sl]

                s = x * x + y * y
                inv = plsc.bitcast(
                    jnp.int32(0x5F3759DF) - (plsc.bitcast(s, jnp.int32) >> 1),
                    jnp.float32)
                h = -0.5 * s
                inv = inv * (1.5 + h * inv * inv)
                inv = inv * (1.5 + h * inv * inv)
                inv = inv * (1.5 + h * inv * inv)

                u = (s * inv) * 8.0
                i0 = u.astype(jnp.int32)
                fi = i0.astype(jnp.float32)
                s64 = s * 64.0
                fh = fi + 1.0
                i1 = jnp.where(s64 >= fh * fh, i0 + 1, i0)
                i1 = jnp.where(s64 < fi * fi, i1 - 1, i1)
                w = jnp.where(i1 < _R_BINS, m, 0.0)
                ic = jnp.minimum(i1, _R_BINS - 1)

                nr = x * vx + y * vy
                nphi = y * vx - x * vy
                vr = nr * inv
                vphi = nphi * inv
                wvr = w * vr
                wvphi = w * vphi
                wvz = w * vz
                bidx = ic * _L + lane
                plsc.addupdate_scatter(hist_v, [bidx], w)
                plsc.addupdate_scatter(hist_v, [bidx + 512], wvr)
                plsc.addupdate_scatter(hist_v, [bidx + 1024], wvr * vr)
                plsc.addupdate_scatter(hist_v, [bidx + 1536], wvphi)
                plsc.addupdate_scatter(hist_v, [bidx + 2048], wvphi * vphi)
                plsc.addupdate_scatter(hist_v, [bidx + 2560], wvz)
                plsc.addupdate_scatter(hist_v, [bidx + 3072], wvz * vz)
                return gcarry

            lax.fori_loop(0, _GROUPS, group_body, 0)

        return carry

    lax.fori_loop(0, _CPW, chunk_body, 0)

    lidx = lane * _L
    for j in range(_ROWS // _L):
        b = j * (_L * _L)
        accv = zero16
        for l in range(_L):
            accv = accv + plsc.load_gather(hist_v, [lidx + (b + l)])
        acc_v[pl.ds(j * _L, _L)] = accv

    pltpu.sync_copy(acc_v, out_hbm.at[wid])


_sc_hist = functools.partial(
    pl.kernel,
    out_type=jax.ShapeDtypeStruct((_NW, _ROWS), jnp.float32),
    mesh=plsc.VectorSubcoreMesh(
        core_axis_name="c", subcore_axis_name="s",
        num_cores=_NC, num_subcores=_NS),
    compiler_params=pltpu.CompilerParams(needs_layout_passes=False),
    scratch_types=[
        pltpu.VMEM((_CHUNK,), jnp.float32),
        pltpu.VMEM((_CHUNK,), jnp.float32),
        pltpu.VMEM((_CHUNK,), jnp.float32),
        pltpu.VMEM((_CHUNK,), jnp.float32),
        pltpu.VMEM((_CHUNK,), jnp.float32),
        pltpu.VMEM((_CHUNK,), jnp.float32),
        pltpu.VMEM((_HIST,), jnp.float32),
        pltpu.VMEM((_ROWS,), jnp.float32),
        pltpu.SemaphoreType.DMA,
    ],
)(_sc_body)


def _epi_body(p_ref, o_ref):
    s = jnp.sum(p_ref[:], axis=0)
    mass = s[0:1, :]
    vr = s[1:2] / mass
    vr2 = s[2:3] / mass
    vphi = s[3:4] / mass
    vphi2 = s[4:5] / mass
    vz = s[5:6] / mass
    vz2 = s[6:7] / mass
    o_ref[:] = jnp.concatenate([
        vphi, jnp.sqrt(vphi2 - vphi * vphi),
        vr, jnp.sqrt(vr2 - vr * vr),
        vz, jnp.sqrt(vz2 - vz * vz)], axis=0)


def kernel(positions, velocities, masses):
    x = positions[:, 0]
    y = positions[:, 1]
    vx = velocities[:, 0]
    vy = velocities[:, 1]
    vz = velocities[:, 2]
    partials = _sc_hist(x, y, vx, vy, vz, masses)
    p3 = partials.reshape(_NW, _NVAL, _R_BINS)
    return pl.pallas_call(
        _epi_body,
        out_shape=jax.ShapeDtypeStruct((6, _R_BINS), jnp.float32),
    )(p3)


# speedup vs baseline: 60.2502x; 1.4192x over previous
import functools

import jax
import jax.numpy as jnp
from jax import lax
from jax.experimental import pallas as pl
from jax.experimental.pallas import tpu as pltpu
from jax.experimental.pallas import tpu_sc as plsc

_R_BINS = 32
_N = 4_000_000
_NC, _NS, _L = 2, 16, 16
_NW = _NC * _NS
_CHUNK = 8000
_NCHUNKS = _N // _CHUNK
_CPW = (_NCHUNKS + _NW - 1) // _NW
_GROUPS = _CHUNK // _L
_NVAL = 7
_HIST = _NVAL * _R_BINS * _L
_ROWS = _NVAL * _R_BINS


def _sc_body(x_hbm, y_hbm, vx_hbm, vy_hbm, vz_hbm, m_hbm, out_hbm,
             x_v, y_v, vx_v, vy_v, vz_v, m_v, hist_v, acc_v, sem):
    cid = lax.axis_index("c")
    sid = lax.axis_index("s")
    wid = sid * _NC + cid

    lane = lax.iota(jnp.int32, _L)
    zero16 = jnp.zeros((_L,), jnp.float32)

    def _zero(j, carry):
        hist_v[pl.ds(j * _L, _L)] = zero16
        return carry

    lax.fori_loop(0, _HIST // _L, _zero, 0)

    def chunk_body(t, carry):
        c = wid + t * _NW

        @pl.when(c < _NCHUNKS)
        def _():
            base = c * _CHUNK
            cps = [
                pltpu.async_copy(h.at[pl.ds(base, _CHUNK)], v, sem)
                for h, v in ((x_hbm, x_v), (y_hbm, y_v), (vx_hbm, vx_v),
                             (vy_hbm, vy_v), (vz_hbm, vz_v), (m_hbm, m_v))
            ]
            for cp in cps:
                cp.wait()

            @plsc.parallel_loop(0, _GROUPS, unroll=4)
            def group_body(g):
                sl = pl.ds(g * _L, _L)
                x = x_v[sl]
                y = y_v[sl]
                vx = vx_v[sl]
                vy = vy_v[sl]
                vz = vz_v[sl]
                m = m_v[sl]

                s = x * x + y * y
                inv = plsc.bitcast(
                    jnp.int32(0x5F3759DF) - (plsc.bitcast(s, jnp.int32) >> 1),
                    jnp.float32)
                h = -0.5 * s
                inv = inv * (1.5 + h * inv * inv)
                inv = inv * (1.5 + h * inv * inv)
                inv = inv * (1.5 + h * inv * inv)

                u = (s * inv) * 8.0
                i0 = u.astype(jnp.int32)
                fi = i0.astype(jnp.float32)
                s64 = s * 64.0
                fh = fi + 1.0
                i1 = jnp.where(s64 >= fh * fh, i0 + 1, i0)
                i1 = jnp.where(s64 < fi * fi, i1 - 1, i1)
                w = jnp.where(i1 < _R_BINS, m, 0.0)
                ic = jnp.minimum(i1, _R_BINS - 1)

                nr = x * vx + y * vy
                nphi = y * vx - x * vy
                vr = nr * inv
                vphi = nphi * inv
                wvr = w * vr
                wvphi = w * vphi
                wvz = w * vz
                bidx = ic * _L + lane
                plsc.addupdate_scatter(hist_v, [bidx], w)
                plsc.addupdate_scatter(hist_v, [bidx + 512], wvr)
                plsc.addupdate_scatter(hist_v, [bidx + 1024], wvr * vr)
                plsc.addupdate_scatter(hist_v, [bidx + 1536], wvphi)
                plsc.addupdate_scatter(hist_v, [bidx + 2048], wvphi * vphi)
                plsc.addupdate_scatter(hist_v, [bidx + 2560], wvz)
                plsc.addupdate_scatter(hist_v, [bidx + 3072], wvz * vz)

        return carry

    lax.fori_loop(0, _CPW, chunk_body, 0)

    lidx = lane * _L
    for j in range(_ROWS // _L):
        b = j * (_L * _L)
        accv = zero16
        for l in range(_L):
            accv = accv + plsc.load_gather(hist_v, [lidx + (b + l)])
        acc_v[pl.ds(j * _L, _L)] = accv

    pltpu.sync_copy(acc_v, out_hbm.at[wid])


_sc_hist = functools.partial(
    pl.kernel,
    out_type=jax.ShapeDtypeStruct((_NW, _ROWS), jnp.float32),
    mesh=plsc.VectorSubcoreMesh(
        core_axis_name="c", subcore_axis_name="s",
        num_cores=_NC, num_subcores=_NS),
    compiler_params=pltpu.CompilerParams(needs_layout_passes=False),
    scratch_types=[
        pltpu.VMEM((_CHUNK,), jnp.float32),
        pltpu.VMEM((_CHUNK,), jnp.float32),
        pltpu.VMEM((_CHUNK,), jnp.float32),
        pltpu.VMEM((_CHUNK,), jnp.float32),
        pltpu.VMEM((_CHUNK,), jnp.float32),
        pltpu.VMEM((_CHUNK,), jnp.float32),
        pltpu.VMEM((_HIST,), jnp.float32),
        pltpu.VMEM((_ROWS,), jnp.float32),
        pltpu.SemaphoreType.DMA,
    ],
)(_sc_body)


def _epi_body(p_ref, o_ref):
    s = jnp.sum(p_ref[:], axis=0)
    mass = s[0:1, :]
    vr = s[1:2] / mass
    vr2 = s[2:3] / mass
    vphi = s[3:4] / mass
    vphi2 = s[4:5] / mass
    vz = s[5:6] / mass
    vz2 = s[6:7] / mass
    o_ref[:] = jnp.concatenate([
        vphi, jnp.sqrt(vphi2 - vphi * vphi),
        vr, jnp.sqrt(vr2 - vr * vr),
        vz, jnp.sqrt(vz2 - vz * vz)], axis=0)


def kernel(positions, velocities, masses):
    x = positions[:, 0]
    y = positions[:, 1]
    vx = velocities[:, 0]
    vy = velocities[:, 1]
    vz = velocities[:, 2]
    partials = _sc_hist(x, y, vx, vy, vz, masses)
    p3 = partials.reshape(_NW, _NVAL, _R_BINS)
    return pl.pallas_call(
        _epi_body,
        out_shape=jax.ShapeDtypeStruct((6, _R_BINS), jnp.float32),
    )(p3)
